# Initial kernel scaffold; baseline (speedup 1.0000x reference)
import jax
import jax.numpy as jnp
from jax.experimental import pallas as pl


def kernel(v, edge_index, W1, b1, W2, b2, W3, b3, Wmu, bmu, Wstd, bstd):
    raise NotImplementedError("write your pallas kernel here")


# speedup vs baseline: 4.1725x; 4.1725x over previous
import functools

import jax
import jax.numpy as jnp
from jax import lax
from jax.experimental import pallas as pl
from jax.experimental.pallas import tpu as pltpu
from jax.experimental.pallas import tpu_sc as plsc

N = 10000
E = 320000
F0 = 128
NPAD = 10240
NW = 32
EPW = NPAD
BLK = 128
NBLK = EPW // BLK
TILE_ROWS = NPAD // 16
RB = 1024
GRID = NPAD // RB

@functools.cache
def _sc_spmm_call():
    mesh = plsc.VectorSubcoreMesh(core_axis_name="c", subcore_axis_name="s")

    @functools.partial(
        pl.kernel,
        out_type=jax.ShapeDtypeStruct((2, NPAD, F0), jnp.float32),
        mesh=mesh,
        scratch_types=[
            pltpu.VMEM((NBLK, BLK), jnp.int32),
            pltpu.VMEM((NBLK, BLK), jnp.int32),
            pltpu.VMEM((BLK, F0), jnp.float32),
            pltpu.VMEM_SHARED((NPAD, F0), jnp.float32),
            pltpu.SemaphoreType.DMA,
        ],
    )
    def sc_spmm(table_h, init0_h, zeros_h, src_h, dst_h, out_h,
                src_v, dst_v, rows_v, acc, sem):
        c = lax.axis_index("c")
        s = lax.axis_index("s")
        wid = c * 16 + s
        base = s * TILE_ROWS

        @pl.when(c == 0)
        def _():
            pltpu.sync_copy(init0_h.at[pl.ds(base, TILE_ROWS)],
                            acc.at[pl.ds(base, TILE_ROWS)])

        @pl.when(c != 0)
        def _():
            pltpu.sync_copy(zeros_h.at[pl.ds(base, TILE_ROWS)],
                            acc.at[pl.ds(base, TILE_ROWS)])

        pltpu.sync_copy(src_h.at[wid], src_v)
        pltpu.sync_copy(dst_h.at[wid], dst_v)
        plsc.subcore_barrier()

        def _edge_block(b, carry):
            pltpu.async_copy(table_h.at[src_v.at[b]], rows_v, sem).wait()
            pltpu.sync_copy(rows_v, acc.at[dst_v.at[b]], add=True)
            return carry

        lax.fori_loop(0, NBLK, _edge_block, 0)
        plsc.subcore_barrier()
        pltpu.sync_copy(acc.at[pl.ds(base, TILE_ROWS)],
                        out_h.at[c, pl.ds(base, TILE_ROWS)])

    return sc_spmm


def _sc_spmm(table, init0, zeros, src, dst):
    return _sc_spmm_call()(table, init0, zeros, src, dst)


def _tc_prep(pdeg, v_pad):
    def body(p_ref, v_ref, dis_ref, xs_ref):
        deg = p_ref[0] + p_ref[1]
        d = lax.rsqrt(deg)
        d = d * (1.5 - 0.5 * deg * d * d)
        dis_ref[...] = d
        xs_ref[...] = d * v_ref[...]

    return pl.pallas_call(
        body, grid=(GRID,),
        in_specs=[pl.BlockSpec((2, RB, F0), lambda i: (0, i, 0)),
                  pl.BlockSpec((RB, F0), lambda i: (i, 0))],
        out_specs=[pl.BlockSpec((RB, F0), lambda i: (i, 0))] * 2,
        out_shape=[jax.ShapeDtypeStruct((NPAD, F0), jnp.float32)] * 2,
    )(pdeg, v_pad)


def _tc_layer(p_list, dis, W, b):
    cin = len(p_list)
    fout = W.shape[1]
    cout = fout // F0
    Wr = W.reshape(cin, F0, fout)
    br = b.reshape(1, fout)

    def body(*refs):
        p_refs = refs[:cin]
        dis_ref, w_ref, b_ref = refs[cin:cin + 3]
        out_refs = refs[cin + 3:]
        d = dis_ref[...]
        acc = jnp.zeros((RB, fout), jnp.float32)
        for cc in range(cin):
            y = (p_refs[cc][0] + p_refs[cc][1]) * d
            acc = acc + jnp.dot(y, w_ref[cc], preferred_element_type=jnp.float32)
        h = jnp.maximum(acc + b_ref[...], 0.0) * d[:, 0:1]
        for k in range(cout):
            out_refs[k][...] = h[:, k * F0:(k + 1) * F0]

    rblk = lambda i: (i, 0)
    in_specs = ([pl.BlockSpec((2, RB, F0), lambda i: (0, i, 0))] * cin
                + [pl.BlockSpec((RB, F0), rblk),
                   pl.BlockSpec((cin, F0, fout), lambda i: (0, 0, 0)),
                   pl.BlockSpec((1, fout), lambda i: (0, 0))])
    out = pl.pallas_call(
        body, grid=(GRID,),
        in_specs=in_specs,
        out_specs=[pl.BlockSpec((RB, F0), rblk)] * cout,
        out_shape=[jax.ShapeDtypeStruct((NPAD, F0), jnp.float32)] * cout,
    )(*p_list, dis, Wr, br)
    return list(out)


def _tc_final(p_list, dis, Wmu, bmu, Wstd, bstd):
    cin = len(p_list)
    fout = Wmu.shape[1]
    Wmur = Wmu.reshape(cin, F0, fout)
    Wstdr = Wstd.reshape(cin, F0, fout)
    bmur = bmu.reshape(1, fout)
    bstdr = bstd.reshape(1, fout)

    def body(*refs):
        p_refs = refs[:cin]
        dis_ref, wmu_ref, bmu_ref, wstd_ref, bstd_ref = refs[cin:cin + 5]
        mu_ref, std_ref = refs[cin + 5:]
        d = dis_ref[...]
        accmu = jnp.zeros((RB, fout), jnp.float32)
        accstd = jnp.zeros((RB, fout), jnp.float32)
        for cc in range(cin):
            t = (p_refs[cc][0] + p_refs[cc][1]) * d
            accmu = accmu + jnp.dot(t, wmu_ref[cc], preferred_element_type=jnp.float32)
            accstd = accstd + jnp.dot(t, wstd_ref[cc], preferred_element_type=jnp.float32)
        mu_ref[...] = accmu + bmu_ref[...]
        std_ref[...] = accstd + bstd_ref[...]

    rblk = lambda i: (i, 0)
    in_specs = ([pl.BlockSpec((2, RB, F0), lambda i: (0, i, 0))] * cin
                + [pl.BlockSpec((RB, F0), rblk),
                   pl.BlockSpec((cin, F0, fout), lambda i: (0, 0, 0)),
                   pl.BlockSpec((1, fout), lambda i: (0, 0)),
                   pl.BlockSpec((cin, F0, fout), lambda i: (0, 0, 0)),
                   pl.BlockSpec((1, fout), lambda i: (0, 0))])
    return pl.pallas_call(
        body, grid=(GRID,),
        in_specs=in_specs,
        out_specs=[pl.BlockSpec((RB, fout), rblk)] * 2,
        out_shape=[jax.ShapeDtypeStruct((NPAD, fout), jnp.float32)] * 2,
    )(*p_list, dis, Wmur, bmur, Wstdr, bstdr)


def kernel(v, edge_index, W1, b1, W2, b2, W3, b3, Wmu, bmu, Wstd, bstd):
    epw = E // NW
    src = edge_index[0].reshape(NW, epw)
    dst = edge_index[1].reshape(NW, epw)
    pad = EPW - epw
    src = jnp.pad(src, ((0, 0), (0, pad))).reshape(NW, NBLK, BLK)
    dst = jnp.pad(dst, ((0, 0), (0, pad)), constant_values=N).reshape(NW, NBLK, BLK)
    zeros = jnp.zeros((NPAD, F0), jnp.float32)
    ones = jnp.ones((NPAD, F0), jnp.float32)
    v_pad = jnp.pad(v, ((0, NPAD - N), (0, 0)))

    pdeg = _sc_spmm(ones, ones, zeros, src, dst)
    dis, xs0 = _tc_prep(pdeg, v_pad)
    xs = [xs0]
    for W, b in ((W1, b1), (W2, b2), (W3, b3)):
        p = [_sc_spmm(ch, ch, zeros, src, dst) for ch in xs]
        xs = _tc_layer(p, dis, W, b)
    p = [_sc_spmm(ch, ch, zeros, src, dst) for ch in xs]
    mu, std = _tc_final(p, dis, Wmu, bmu, Wstd, bstd)
    return mu[:N], std[:N]


# speedup vs baseline: 4.4428x; 1.0648x over previous
import functools

import jax
import jax.numpy as jnp
from jax import lax
from jax.experimental import pallas as pl
from jax.experimental.pallas import tpu as pltpu
from jax.experimental.pallas import tpu_sc as plsc

N = 10000
E = 320000
F0 = 128
NPAD = 10112
NW = 32
EPW = 10240
BLK = 128
NBLK = EPW // BLK
WND = 40
TILE_ROWS = NPAD // 16
RB = NPAD // 8
GRID = NPAD // RB
NBUF = 2


@functools.cache
def _sc_spmm_call():
    mesh = plsc.VectorSubcoreMesh(core_axis_name="c", subcore_axis_name="s")

    @functools.partial(
        pl.kernel,
        out_type=jax.ShapeDtypeStruct((2, NPAD, F0), jnp.float32),
        mesh=mesh,
        scratch_types=[
            pltpu.VMEM((WND, BLK), jnp.int32),
            pltpu.VMEM((WND, BLK), jnp.int32),
            pltpu.VMEM((NBUF, BLK, F0), jnp.float32),
            pltpu.VMEM_SHARED((NPAD, F0), jnp.float32),
            pltpu.SemaphoreType.DMA((NBUF,)),
            pltpu.SemaphoreType.DMA((NBUF,)),
        ],
    )
    def sc_spmm(table_h, init0_h, zeros_h, src_h, dst_h, out_h,
                src_v, dst_v, rows_v, acc, gsem, ssem):
        c = lax.axis_index("c")
        s = lax.axis_index("s")
        wid = c * 16 + s
        base = s * TILE_ROWS

        @pl.when(c == 0)
        def _():
            pltpu.sync_copy(init0_h.at[pl.ds(base, TILE_ROWS)],
                            acc.at[pl.ds(base, TILE_ROWS)])

        @pl.when(c != 0)
        def _():
            pltpu.sync_copy(zeros_h.at[pl.ds(base, TILE_ROWS)],
                            acc.at[pl.ds(base, TILE_ROWS)])

        plsc.subcore_barrier()

        for ph in range(NBLK // WND):
            pltpu.sync_copy(src_h.at[wid, pl.ds(ph * WND, WND)], src_v)
            pltpu.sync_copy(dst_h.at[wid, pl.ds(ph * WND, WND)], dst_v)

            for b in range(NBUF):
                pltpu.async_copy(table_h.at[src_v.at[b]], rows_v.at[b],
                                 gsem.at[b])

            def _edge_group(grp, carry):
                g0 = grp * NBUF
                for b in range(NBUF):
                    g = g0 + b
                    pltpu.make_async_copy(table_h.at[src_v.at[g]],
                                          rows_v.at[b], gsem.at[b]).wait()
                    pltpu.async_copy(rows_v.at[b], acc.at[dst_v.at[g]],
                                     ssem.at[b], add=True)
                for b in range(NBUF):
                    g = g0 + b

                    @pl.when(g + NBUF < WND)
                    def _():
                        pltpu.make_async_copy(rows_v.at[b],
                                              acc.at[dst_v.at[g]],
                                              ssem.at[b]).wait()
                        pltpu.async_copy(table_h.at[src_v.at[g + NBUF]],
                                         rows_v.at[b], gsem.at[b])
                return carry

            lax.fori_loop(0, WND // NBUF, _edge_group, 0)
            for b in range(NBUF):
                g = WND - NBUF + b
                pltpu.make_async_copy(rows_v.at[b], acc.at[dst_v.at[g]],
                                      ssem.at[b]).wait()

        plsc.subcore_barrier()
        pltpu.sync_copy(acc.at[pl.ds(base, TILE_ROWS)],
                        out_h.at[c, pl.ds(base, TILE_ROWS)])

    return sc_spmm


def _sc_spmm(table, init0, zeros, src, dst):
    return _sc_spmm_call()(table, init0, zeros, src, dst)


def _tc_prep(pdeg, v_pad):
    def body(p_ref, v_ref, dis_ref, xs_ref):
        deg = p_ref[0] + p_ref[1]
        d = lax.rsqrt(deg)
        d = d * (1.5 - 0.5 * deg * d * d)
        dis_ref[...] = d
        xs_ref[...] = d * v_ref[...]

    return pl.pallas_call(
        body, grid=(GRID,),
        in_specs=[pl.BlockSpec((2, RB, F0), lambda i: (0, i, 0)),
                  pl.BlockSpec((RB, F0), lambda i: (i, 0))],
        out_specs=[pl.BlockSpec((RB, F0), lambda i: (i, 0))] * 2,
        out_shape=[jax.ShapeDtypeStruct((NPAD, F0), jnp.float32)] * 2,
    )(pdeg, v_pad)


def _tc_layer(p_list, dis, W, b):
    cin = len(p_list)
    fout = W.shape[1]
    cout = fout // F0
    Wr = W.reshape(cin, F0, fout)
    br = b.reshape(1, fout)

    def body(*refs):
        p_refs = refs[:cin]
        dis_ref, w_ref, b_ref = refs[cin:cin + 3]
        out_refs = refs[cin + 3:]
        d = dis_ref[...]
        acc = jnp.zeros((RB, fout), jnp.float32)
        for cc in range(cin):
            y = (p_refs[cc][0] + p_refs[cc][1]) * d
            acc = acc + jnp.dot(y, w_ref[cc], preferred_element_type=jnp.float32)
        h = jnp.maximum(acc + b_ref[...], 0.0) * d[:, 0:1]
        for k in range(cout):
            out_refs[k][...] = h[:, k * F0:(k + 1) * F0]

    rblk = lambda i: (i, 0)
    in_specs = ([pl.BlockSpec((2, RB, F0), lambda i: (0, i, 0))] * cin
                + [pl.BlockSpec((RB, F0), rblk),
                   pl.BlockSpec((cin, F0, fout), lambda i: (0, 0, 0)),
                   pl.BlockSpec((1, fout), lambda i: (0, 0))])
    out = pl.pallas_call(
        body, grid=(GRID,),
        in_specs=in_specs,
        out_specs=[pl.BlockSpec((RB, F0), rblk)] * cout,
        out_shape=[jax.ShapeDtypeStruct((NPAD, F0), jnp.float32)] * cout,
    )(*p_list, dis, Wr, br)
    return list(out)


def _tc_final(p_list, dis, Wmu, bmu, Wstd, bstd):
    cin = len(p_list)
    fout = Wmu.shape[1]
    Wmur = Wmu.reshape(cin, F0, fout)
    Wstdr = Wstd.reshape(cin, F0, fout)
    bmur = bmu.reshape(1, fout)
    bstdr = bstd.reshape(1, fout)

    def body(*refs):
        p_refs = refs[:cin]
        dis_ref, wmu_ref, bmu_ref, wstd_ref, bstd_ref = refs[cin:cin + 5]
        mu_ref, std_ref = refs[cin + 5:]
        d = dis_ref[...]
        accmu = jnp.zeros((RB, fout), jnp.float32)
        accstd = jnp.zeros((RB, fout), jnp.float32)
        for cc in range(cin):
            t = (p_refs[cc][0] + p_refs[cc][1]) * d
            accmu = accmu + jnp.dot(t, wmu_ref[cc], preferred_element_type=jnp.float32)
            accstd = accstd + jnp.dot(t, wstd_ref[cc], preferred_element_type=jnp.float32)
        mu_ref[...] = accmu + bmu_ref[...]
        std_ref[...] = accstd + bstd_ref[...]

    rblk = lambda i: (i, 0)
    in_specs = ([pl.BlockSpec((2, RB, F0), lambda i: (0, i, 0))] * cin
                + [pl.BlockSpec((RB, F0), rblk),
                   pl.BlockSpec((cin, F0, fout), lambda i: (0, 0, 0)),
                   pl.BlockSpec((1, fout), lambda i: (0, 0)),
                   pl.BlockSpec((cin, F0, fout), lambda i: (0, 0, 0)),
                   pl.BlockSpec((1, fout), lambda i: (0, 0))])
    return pl.pallas_call(
        body, grid=(GRID,),
        in_specs=in_specs,
        out_specs=[pl.BlockSpec((RB, fout), rblk)] * 2,
        out_shape=[jax.ShapeDtypeStruct((NPAD, fout), jnp.float32)] * 2,
    )(*p_list, dis, Wmur, bmur, Wstdr, bstdr)


def kernel(v, edge_index, W1, b1, W2, b2, W3, b3, Wmu, bmu, Wstd, bstd):
    epw = E // NW
    src = edge_index[0].reshape(NW, epw)
    dst = edge_index[1].reshape(NW, epw)
    pad = EPW - epw
    src = jnp.pad(src, ((0, 0), (0, pad))).reshape(NW, NBLK, BLK)
    dst = jnp.pad(dst, ((0, 0), (0, pad)), constant_values=N).reshape(NW, NBLK, BLK)
    zeros = jnp.zeros((NPAD, F0), jnp.float32)
    ones = jnp.ones((NPAD, F0), jnp.float32)
    v_pad = jnp.pad(v, ((0, NPAD - N), (0, 0)))

    pdeg = _sc_spmm(ones, ones, zeros, src, dst)
    dis, xs0 = _tc_prep(pdeg, v_pad)
    xs = [xs0]
    for W, b in ((W1, b1), (W2, b2), (W3, b3)):
        p = [_sc_spmm(ch, ch, zeros, src, dst) for ch in xs]
        xs = _tc_layer(p, dis, W, b)
    p = [_sc_spmm(ch, ch, zeros, src, dst) for ch in xs]
    mu, std = _tc_final(p, dis, Wmu, bmu, Wstd, bstd)
    return mu[:N], std[:N]


# speedup vs baseline: 4.7473x; 1.0685x over previous
import functools

import jax
import jax.numpy as jnp
from jax import lax
from jax.experimental import pallas as pl
from jax.experimental.pallas import tpu as pltpu
from jax.experimental.pallas import tpu_sc as plsc

N = 10000
E = 320000
F0 = 128
NPAD = 10112
NW = 32
EPW = 10240
BLK = 128
NBLK = EPW // BLK
WND = 40
TILE_ROWS = NPAD // 16
RB = NPAD // 8
GRID = NPAD // RB
NBUF = 2


@functools.cache
def _sc_spmm_call(nch):
    mesh = plsc.VectorSubcoreMesh(core_axis_name="c", subcore_axis_name="s")

    @functools.partial(
        pl.kernel,
        out_type=jax.ShapeDtypeStruct((nch, 2, NPAD, F0), jnp.float32),
        mesh=mesh,
        scratch_types=[
            pltpu.VMEM((WND, BLK), jnp.int32),
            pltpu.VMEM((WND, BLK), jnp.int32),
            pltpu.VMEM((NBUF, BLK, F0), jnp.float32),
            pltpu.VMEM_SHARED((NPAD, F0), jnp.float32),
            pltpu.SemaphoreType.DMA((NBUF,)),
            pltpu.SemaphoreType.DMA((NBUF,)),
        ],
    )
    def sc_spmm(table_h, zeros_h, src_h, dst_h, out_h,
                src_v, dst_v, rows_v, acc, gsem, ssem):
        c = lax.axis_index("c")
        s = lax.axis_index("s")
        wid = c * 16 + s
        base = s * TILE_ROWS

        for ch in range(nch):
            tbl = table_h.at[ch]

            @pl.when(c == 0)
            def _():
                pltpu.sync_copy(tbl.at[pl.ds(base, TILE_ROWS)],
                                acc.at[pl.ds(base, TILE_ROWS)])

            @pl.when(c != 0)
            def _():
                pltpu.sync_copy(zeros_h.at[pl.ds(base, TILE_ROWS)],
                                acc.at[pl.ds(base, TILE_ROWS)])

            plsc.subcore_barrier()

            for ph in range(NBLK // WND):
                pltpu.sync_copy(src_h.at[wid, pl.ds(ph * WND, WND)], src_v)
                pltpu.sync_copy(dst_h.at[wid, pl.ds(ph * WND, WND)], dst_v)

                for b in range(NBUF):
                    pltpu.async_copy(tbl.at[src_v.at[b]], rows_v.at[b],
                                     gsem.at[b])

                def _edge_group(grp, carry):
                    g0 = grp * NBUF
                    for b in range(NBUF):
                        g = g0 + b
                        pltpu.make_async_copy(tbl.at[src_v.at[g]],
                                              rows_v.at[b], gsem.at[b]).wait()
                        pltpu.async_copy(rows_v.at[b], acc.at[dst_v.at[g]],
                                         ssem.at[b], add=True)
                    for b in range(NBUF):
                        g = g0 + b

                        @pl.when(g + NBUF < WND)
                        def _():
                            pltpu.make_async_copy(rows_v.at[b],
                                                  acc.at[dst_v.at[g]],
                                                  ssem.at[b]).wait()
                            pltpu.async_copy(tbl.at[src_v.at[g + NBUF]],
                                             rows_v.at[b], gsem.at[b])
                    return carry

                lax.fori_loop(0, WND // NBUF, _edge_group, 0)
                for b in range(NBUF):
                    g = WND - NBUF + b
                    pltpu.make_async_copy(rows_v.at[b], acc.at[dst_v.at[g]],
                                          ssem.at[b]).wait()

            plsc.subcore_barrier()
            pltpu.sync_copy(acc.at[pl.ds(base, TILE_ROWS)],
                            out_h.at[ch, c, pl.ds(base, TILE_ROWS)])

    return sc_spmm


@functools.cache
def _sc_deg_call():
    mesh = plsc.VectorSubcoreMesh(core_axis_name="c", subcore_axis_name="s")

    @functools.partial(
        pl.kernel,
        out_type=jax.ShapeDtypeStruct((2, NPAD, F0), jnp.float32),
        mesh=mesh,
        scratch_types=[
            pltpu.VMEM((WND, BLK), jnp.int32),
            pltpu.VMEM((BLK, F0), jnp.float32),
            pltpu.VMEM_SHARED((NPAD, F0), jnp.float32),
            pltpu.SemaphoreType.DMA((NBUF,)),
        ],
    )
    def sc_deg(ones_h, zeros_h, dst_h, out_h, dst_v, ones_v, acc, ssem):
        c = lax.axis_index("c")
        s = lax.axis_index("s")
        wid = c * 16 + s
        base = s * TILE_ROWS

        @pl.when(c == 0)
        def _():
            pltpu.sync_copy(ones_h.at[pl.ds(base, TILE_ROWS)],
                            acc.at[pl.ds(base, TILE_ROWS)])

        @pl.when(c != 0)
        def _():
            pltpu.sync_copy(zeros_h.at[pl.ds(base, TILE_ROWS)],
                            acc.at[pl.ds(base, TILE_ROWS)])

        pltpu.sync_copy(ones_h.at[pl.ds(0, BLK)], ones_v)
        plsc.subcore_barrier()

        for ph in range(NBLK // WND):
            pltpu.sync_copy(dst_h.at[wid, pl.ds(ph * WND, WND)], dst_v)
            for b in range(NBUF):
                pltpu.async_copy(ones_v, acc.at[dst_v.at[b]], ssem.at[b],
                                 add=True)

            def _grp(grp, carry):
                g0 = grp * NBUF
                for b in range(NBUF):
                    g = g0 + b

                    @pl.when(g + NBUF < WND)
                    def _():
                        pltpu.make_async_copy(ones_v, acc.at[dst_v.at[g]],
                                              ssem.at[b]).wait()
                        pltpu.async_copy(ones_v, acc.at[dst_v.at[g + NBUF]],
                                         ssem.at[b], add=True)
                return carry

            lax.fori_loop(0, WND // NBUF, _grp, 0)
            for b in range(NBUF):
                g = WND - NBUF + b
                pltpu.make_async_copy(ones_v, acc.at[dst_v.at[g]],
                                      ssem.at[b]).wait()

        plsc.subcore_barrier()
        pltpu.sync_copy(acc.at[pl.ds(base, TILE_ROWS)],
                        out_h.at[c, pl.ds(base, TILE_ROWS)])

    return sc_deg


def _sc_spmm(tables, zeros, src, dst):
    return _sc_spmm_call(tables.shape[0])(tables, zeros, src, dst)


def _sc_deg(ones, zeros, dst):
    return _sc_deg_call()(ones, zeros, dst)


def _tc_prep(pdeg, v_pad):
    def body(p_ref, v_ref, dis_ref, xs_ref):
        deg = p_ref[0] + p_ref[1]
        d = lax.rsqrt(deg)
        d = d * (1.5 - 0.5 * deg * d * d)
        dis_ref[...] = d
        xs_ref[...] = d * v_ref[...]

    return pl.pallas_call(
        body, grid=(GRID,),
        in_specs=[pl.BlockSpec((2, RB, F0), lambda i: (0, i, 0)),
                  pl.BlockSpec((RB, F0), lambda i: (i, 0))],
        out_specs=[pl.BlockSpec((RB, F0), lambda i: (i, 0))] * 2,
        out_shape=[jax.ShapeDtypeStruct((NPAD, F0), jnp.float32)] * 2,
    )(pdeg, v_pad)


def _tc_layer(p, dis, W, b):
    cin = p.shape[0]
    fout = W.shape[1]
    cout = fout // F0
    Wr = W.reshape(cin, F0, fout)
    br = b.reshape(1, fout)

    def body(p_ref, dis_ref, w_ref, b_ref, out_ref):
        d = dis_ref[...]
        acc = jnp.zeros((RB, fout), jnp.float32)
        for cc in range(cin):
            y = (p_ref[cc, 0] + p_ref[cc, 1]) * d
            acc = acc + jnp.dot(y, w_ref[cc], preferred_element_type=jnp.float32)
        h = jnp.maximum(acc + b_ref[...], 0.0) * d[:, 0:1]
        for k in range(cout):
            out_ref[k] = h[:, k * F0:(k + 1) * F0]

    return pl.pallas_call(
        body, grid=(GRID,),
        in_specs=[pl.BlockSpec((cin, 2, RB, F0), lambda i: (0, 0, i, 0)),
                  pl.BlockSpec((RB, F0), lambda i: (i, 0)),
                  pl.BlockSpec((cin, F0, fout), lambda i: (0, 0, 0)),
                  pl.BlockSpec((1, fout), lambda i: (0, 0))],
        out_specs=pl.BlockSpec((cout, RB, F0), lambda i: (0, i, 0)),
        out_shape=jax.ShapeDtypeStruct((cout, NPAD, F0), jnp.float32),
    )(p, dis, Wr, br)


def _tc_final(p, dis, Wmu, bmu, Wstd, bstd):
    cin = p.shape[0]
    fout = Wmu.shape[1]
    Wmur = Wmu.reshape(cin, F0, fout)
    Wstdr = Wstd.reshape(cin, F0, fout)
    bmur = bmu.reshape(1, fout)
    bstdr = bstd.reshape(1, fout)

    def body(p_ref, dis_ref, wmu_ref, bmu_ref, wstd_ref, bstd_ref,
             mu_ref, std_ref):
        d = dis_ref[...]
        accmu = jnp.zeros((RB, fout), jnp.float32)
        accstd = jnp.zeros((RB, fout), jnp.float32)
        for cc in range(cin):
            t = (p_ref[cc, 0] + p_ref[cc, 1]) * d
            accmu = accmu + jnp.dot(t, wmu_ref[cc], preferred_element_type=jnp.float32)
            accstd = accstd + jnp.dot(t, wstd_ref[cc], preferred_element_type=jnp.float32)
        mu_ref[...] = accmu + bmu_ref[...]
        std_ref[...] = accstd + bstd_ref[...]

    rblk = lambda i: (i, 0)
    return pl.pallas_call(
        body, grid=(GRID,),
        in_specs=[pl.BlockSpec((cin, 2, RB, F0), lambda i: (0, 0, i, 0)),
                  pl.BlockSpec((RB, F0), rblk),
                  pl.BlockSpec((cin, F0, fout), lambda i: (0, 0, 0)),
                  pl.BlockSpec((1, fout), lambda i: (0, 0)),
                  pl.BlockSpec((cin, F0, fout), lambda i: (0, 0, 0)),
                  pl.BlockSpec((1, fout), lambda i: (0, 0))],
        out_specs=[pl.BlockSpec((RB, fout), rblk)] * 2,
        out_shape=[jax.ShapeDtypeStruct((NPAD, fout), jnp.float32)] * 2,
    )(p, dis, Wmur, bmur, Wstdr, bstdr)


def kernel(v, edge_index, W1, b1, W2, b2, W3, b3, Wmu, bmu, Wstd, bstd):
    epw = E // NW
    src = edge_index[0].reshape(NW, epw)
    dst = edge_index[1].reshape(NW, epw)
    pad = EPW - epw
    src = jnp.pad(src, ((0, 0), (0, pad))).reshape(NW, NBLK, BLK)
    dst = jnp.pad(dst, ((0, 0), (0, pad)), constant_values=N).reshape(NW, NBLK, BLK)
    zeros = jnp.zeros((NPAD, F0), jnp.float32)
    ones = jnp.ones((NPAD, F0), jnp.float32)
    v_pad = jnp.pad(v, ((0, NPAD - N), (0, 0)))

    pdeg = _sc_deg(ones, zeros, dst)
    dis, xs0 = _tc_prep(pdeg, v_pad)
    xs = xs0[None]
    for W, b in ((W1, b1), (W2, b2), (W3, b3)):
        p = _sc_spmm(xs, zeros, src, dst)
        xs = _tc_layer(p, dis, W, b)
    p = _sc_spmm(xs, zeros, src, dst)
    mu, std = _tc_final(p, dis, Wmu, bmu, Wstd, bstd)
    return mu[:N], std[:N]


# speedup vs baseline: 6.6544x; 1.4017x over previous
import functools

import jax
import jax.numpy as jnp
from jax import lax
from jax.experimental import pallas as pl
from jax.experimental.pallas import tpu as pltpu
from jax.experimental.pallas import tpu_sc as plsc

N = 10000
E = 320000
F0 = 128
NPAD = 10112
NW = 32
EPW = 10240
BLK = 128
NBLK = EPW // BLK
WND = 40
TILE_ROWS = NPAD // 16
RB = NPAD // 8
GRID = NPAD // RB
NBUF = 2


@functools.cache
def _sc_spmm_call(nch):
    mesh = plsc.VectorSubcoreMesh(core_axis_name="c", subcore_axis_name="s")

    @functools.partial(
        pl.kernel,
        out_type=jax.ShapeDtypeStruct((nch, 2, NPAD, F0), jnp.float32),
        mesh=mesh,
        scratch_types=[
            pltpu.VMEM((WND, BLK), jnp.int32),
            pltpu.VMEM((WND, BLK), jnp.int32),
            pltpu.VMEM((NBUF, BLK, F0), jnp.float32),
            pltpu.VMEM_SHARED((NPAD, F0), jnp.float32),
            pltpu.SemaphoreType.DMA((NBUF,)),
            pltpu.SemaphoreType.DMA((NBUF,)),
        ],
    )
    def sc_spmm(table_h, zeros_h, src_h, dst_h, out_h,
                src_v, dst_v, rows_v, acc, gsem, ssem):
        c = lax.axis_index("c")
        s = lax.axis_index("s")
        wid = c * 16 + s
        base = s * TILE_ROWS

        for ch in range(nch):
            tbl = table_h.at[ch]

            @pl.when(c == 0)
            def _():
                pltpu.sync_copy(tbl.at[pl.ds(base, TILE_ROWS)],
                                acc.at[pl.ds(base, TILE_ROWS)])

            @pl.when(c != 0)
            def _():
                pltpu.sync_copy(zeros_h.at[pl.ds(base, TILE_ROWS)],
                                acc.at[pl.ds(base, TILE_ROWS)])

            plsc.subcore_barrier()

            for ph in range(NBLK // WND):
                pltpu.sync_copy(src_h.at[wid, pl.ds(ph * WND, WND)], src_v)
                pltpu.sync_copy(dst_h.at[wid, pl.ds(ph * WND, WND)], dst_v)

                for b in range(NBUF):
                    pltpu.async_copy(tbl.at[src_v.at[b]], rows_v.at[b],
                                     gsem.at[b])

                def _edge_group(grp, carry):
                    g0 = grp * NBUF
                    for b in range(NBUF):
                        g = g0 + b
                        pltpu.make_async_copy(tbl.at[src_v.at[g]],
                                              rows_v.at[b], gsem.at[b]).wait()
                        pltpu.async_copy(rows_v.at[b], acc.at[dst_v.at[g]],
                                         ssem.at[b], add=True)
                    for b in range(NBUF):
                        g = g0 + b

                        @pl.when(g + NBUF < WND)
                        def _():
                            pltpu.make_async_copy(rows_v.at[b],
                                                  acc.at[dst_v.at[g]],
                                                  ssem.at[b]).wait()
                            pltpu.async_copy(tbl.at[src_v.at[g + NBUF]],
                                             rows_v.at[b], gsem.at[b])
                    return carry

                lax.fori_loop(0, WND // NBUF, _edge_group, 0)
                for b in range(NBUF):
                    g = WND - NBUF + b
                    pltpu.make_async_copy(rows_v.at[b], acc.at[dst_v.at[g]],
                                          ssem.at[b]).wait()

            plsc.subcore_barrier()
            pltpu.sync_copy(acc.at[pl.ds(base, TILE_ROWS)],
                            out_h.at[ch, c, pl.ds(base, TILE_ROWS)])

    return sc_spmm


@functools.cache
def _sc_deg_call():
    mesh = plsc.VectorSubcoreMesh(core_axis_name="c", subcore_axis_name="s")

    @functools.partial(
        pl.kernel,
        out_type=jax.ShapeDtypeStruct((2, NPAD, F0), jnp.float32),
        mesh=mesh,
        scratch_types=[
            pltpu.VMEM((WND, BLK), jnp.int32),
            pltpu.VMEM((BLK, F0), jnp.float32),
            pltpu.VMEM_SHARED((NPAD, F0), jnp.float32),
            pltpu.SemaphoreType.DMA((NBUF,)),
        ],
    )
    def sc_deg(ones_h, zeros_h, dst_h, out_h, dst_v, ones_v, acc, ssem):
        c = lax.axis_index("c")
        s = lax.axis_index("s")
        wid = c * 16 + s
        base = s * TILE_ROWS

        @pl.when(c == 0)
        def _():
            pltpu.sync_copy(ones_h.at[pl.ds(base, TILE_ROWS)],
                            acc.at[pl.ds(base, TILE_ROWS)])

        @pl.when(c != 0)
        def _():
            pltpu.sync_copy(zeros_h.at[pl.ds(base, TILE_ROWS)],
                            acc.at[pl.ds(base, TILE_ROWS)])

        pltpu.sync_copy(ones_h.at[pl.ds(0, BLK)], ones_v)
        plsc.subcore_barrier()

        for ph in range(NBLK // WND):
            pltpu.sync_copy(dst_h.at[wid, pl.ds(ph * WND, WND)], dst_v)
            for b in range(NBUF):
                pltpu.async_copy(ones_v, acc.at[dst_v.at[b]], ssem.at[b],
                                 add=True)

            def _grp(grp, carry):
                g0 = grp * NBUF
                for b in range(NBUF):
                    g = g0 + b

                    @pl.when(g + NBUF < WND)
                    def _():
                        pltpu.make_async_copy(ones_v, acc.at[dst_v.at[g]],
                                              ssem.at[b]).wait()
                        pltpu.async_copy(ones_v, acc.at[dst_v.at[g + NBUF]],
                                         ssem.at[b], add=True)
                return carry

            lax.fori_loop(0, WND // NBUF, _grp, 0)
            for b in range(NBUF):
                g = WND - NBUF + b
                pltpu.make_async_copy(ones_v, acc.at[dst_v.at[g]],
                                      ssem.at[b]).wait()

        plsc.subcore_barrier()
        pltpu.sync_copy(acc.at[pl.ds(base, TILE_ROWS)],
                        out_h.at[c, pl.ds(base, TILE_ROWS)])

    return sc_deg


def _sc_spmm(tables, zeros, src, dst):
    return _sc_spmm_call(tables.shape[0])(tables, zeros, src, dst)


def _sc_deg(ones, zeros, dst):
    return _sc_deg_call()(ones, zeros, dst)


HALF = NPAD // 2
NR = 5072
TR2 = NR // 16
BLK2 = 64
NBLK2 = EPW // BLK2
WND2 = 32


@functools.cache
def _sc_partition_call():
    mesh = plsc.VectorSubcoreMesh(core_axis_name="c", subcore_axis_name="s")

    @functools.partial(
        pl.kernel,
        out_type=(jax.ShapeDtypeStruct((NW, 2 * EPW), jnp.int32),
                  jax.ShapeDtypeStruct((NW, 2 * EPW), jnp.int32),
                  jax.ShapeDtypeStruct((NW, 16), jnp.int32)),
        mesh=mesh,
        compiler_params=pltpu.CompilerParams(needs_layout_passes=False),
        scratch_types=[
            pltpu.VMEM((EPW,), jnp.int32),
            pltpu.VMEM((EPW,), jnp.int32),
            pltpu.VMEM((2 * EPW,), jnp.int32),
            pltpu.VMEM((2 * EPW,), jnp.int32),
            pltpu.VMEM((16,), jnp.int32),
        ],
    )
    def sc_part(srcF_h, dstF_h, tsrc_h, tdst_h, psrc_h, pdst_h, pcnt_h,
                sv, dv, lsrc, ldst, cntv):
        c = lax.axis_index("c")
        s = lax.axis_index("s")
        wid = c * 16 + s
        pltpu.sync_copy(srcF_h.at[wid], sv)
        pltpu.sync_copy(dstF_h.at[wid], dv)
        pltpu.sync_copy(tsrc_h, lsrc)
        pltpu.sync_copy(tdst_h, ldst)

        def _scan(i, carry):
            c0, c1 = carry
            s16 = sv[pl.ds(i * 16, 16)]
            d16 = dv[pl.ds(i * 16, 16)]
            m0 = d16 < HALF
            cnts = []
            for r, cr in ((0, c0), (1, c1)):
                m = m0 if r == 0 else jnp.logical_not(m0)
                csum = plsc.cumsum(m.astype(jnp.int32))
                pos = r * EPW + cr + csum - 1
                dloc = d16 - r * HALF
                plsc.store_scatter(lsrc, [pos], s16, mask=m)
                plsc.store_scatter(ldst, [pos], dloc, mask=m)
                cnts.append(cr + jnp.max(csum))
            return tuple(cnts)

        c0, c1 = lax.fori_loop(0, EPW // 16, _scan,
                               (jnp.int32(0), jnp.int32(0)))
        nb0 = (c0 + BLK2 - 1) >> 6
        nb1 = (c1 + BLK2 - 1) >> 6
        ar = jnp.arange(16, dtype=jnp.int32)
        cntv[...] = jnp.where(ar == 0, nb0, jnp.where(ar == 1, nb1, 0))
        pltpu.sync_copy(lsrc, psrc_h.at[wid])
        pltpu.sync_copy(ldst, pdst_h.at[wid])
        pltpu.sync_copy(cntv, pcnt_h.at[wid])

    return sc_part


def _sc_partition(srcF, dstF, tsrc, tdst):
    return _sc_partition_call()(srcF, dstF, tsrc, tdst)


@functools.cache
def _sc_spmm256_call(nch):
    mesh = plsc.VectorSubcoreMesh(core_axis_name="c", subcore_axis_name="s")

    @functools.partial(
        pl.kernel,
        out_type=jax.ShapeDtypeStruct((nch, 2, NR, 2, F0), jnp.float32),
        mesh=mesh,
        scratch_types=[
            pltpu.VMEM((2, WND2, BLK2), jnp.int32),
            pltpu.VMEM((2, WND2, BLK2), jnp.int32),
            pltpu.VMEM((2, BLK2, 2, F0), jnp.float32),
            pltpu.VMEM((16,), jnp.int32),
            pltpu.VMEM_SHARED((NR, 2, F0), jnp.float32),
            pltpu.SemaphoreType.DMA((2,)),
            pltpu.SemaphoreType.DMA((2,)),
        ],
    )
    def sc_spmm256(tables_h, zeros2_h, psrc_h, pdst_h, pcnt_h, out_h,
                   swnd, dwnd, rows_v, cntv, acc, gsem, ssem):
        c = lax.axis_index("c")
        s = lax.axis_index("s")
        base = s * TR2

        for ch in range(nch):
            tbl = tables_h.at[ch]
            pltpu.sync_copy(zeros2_h.at[pl.ds(base, TR2)],
                            acc.at[pl.ds(base, TR2)])
            plsc.subcore_barrier()

            for wi in range(2):
                w = wi * 16 + s
                pltpu.sync_copy(pcnt_h.at[w], cntv)
                cv = cntv[...]
                nb = jnp.where(c == 0, cv[0], cv[1])

                def _stage(wnd):
                    buf = lax.rem(wnd, 2)
                    pltpu.sync_copy(
                        psrc_h.at[w, c, pl.ds(wnd * WND2, WND2)],
                        swnd.at[buf])
                    pltpu.sync_copy(
                        pdst_h.at[w, c, pl.ds(wnd * WND2, WND2)],
                        dwnd.at[buf])

                @pl.when(nb > 0)
                def _():
                    _stage(jnp.int32(0))
                    pltpu.async_copy(tbl.at[swnd.at[0, 0]], rows_v.at[0],
                                     gsem.at[0])

                @pl.when(nb > 1)
                def _():
                    pltpu.async_copy(tbl.at[swnd.at[0, 1]], rows_v.at[1],
                                     gsem.at[1])

                def _pair(grp, carry):
                    for b in range(2):
                        g = grp * 2 + b

                        @pl.when(g < nb)
                        def _():
                            wb = lax.rem(g // WND2, 2)
                            j = lax.rem(g, WND2)
                            pltpu.make_async_copy(tbl.at[swnd.at[wb, j]],
                                                  rows_v.at[b],
                                                  gsem.at[b]).wait()
                            pltpu.async_copy(rows_v.at[b],
                                             acc.at[dwnd.at[wb, j]],
                                             ssem.at[b], add=True)
                            g2 = g + 2

                            @pl.when(g2 < nb)
                            def _():
                                pltpu.make_async_copy(rows_v.at[b],
                                                      acc.at[dwnd.at[wb, j]],
                                                      ssem.at[b]).wait()

                                @pl.when(lax.rem(g2, WND2) == 0)
                                def _():
                                    _stage(g2 // WND2)

                                wb2 = lax.rem(g2 // WND2, 2)
                                j2 = lax.rem(g2, WND2)
                                pltpu.async_copy(tbl.at[swnd.at[wb2, j2]],
                                                 rows_v.at[b], gsem.at[b])
                    return carry

                lax.fori_loop(0, (nb + 1) // 2, _pair, 0)
                for b in range(2):

                    @pl.when(nb >= b + 1)
                    def _():
                        pltpu.make_async_copy(rows_v.at[b],
                                              acc.at[dwnd.at[0, 0]],
                                              ssem.at[b]).wait()

            plsc.subcore_barrier()
            pltpu.sync_copy(acc.at[pl.ds(base, TR2)],
                            out_h.at[ch, c, pl.ds(base, TR2)])

    return sc_spmm256


def _sc_spmm256(tables, zeros2, psrc, pdst, pcnt):
    return _sc_spmm256_call(tables.shape[0])(tables, zeros2, psrc, pdst,
                                             pcnt)


RB2 = 1264


def _tc_layer1(p, dis, W, b):
    fout = W.shape[1]
    br = b.reshape(1, fout)

    def body(p_ref, dis_ref, w_ref, b_ref, out_ref):
        d = dis_ref[...]
        y = (p_ref[0, 0] + p_ref[0, 1]) * d
        acc = jnp.dot(y, w_ref[...], preferred_element_type=jnp.float32)
        h = jnp.maximum(acc + b_ref[...], 0.0) * d[:, 0:1]
        out_ref[0, :, 0, :] = h[:, :F0]
        out_ref[0, :, 1, :] = h[:, F0:]

    return pl.pallas_call(
        body, grid=(GRID,),
        in_specs=[pl.BlockSpec((1, 2, RB, F0), lambda i: (0, 0, i, 0)),
                  pl.BlockSpec((RB, F0), lambda i: (i, 0)),
                  pl.BlockSpec((F0, fout), lambda i: (0, 0)),
                  pl.BlockSpec((1, fout), lambda i: (0, 0))],
        out_specs=pl.BlockSpec((1, RB, 2, F0), lambda i: (0, i, 0, 0)),
        out_shape=jax.ShapeDtypeStruct((1, NPAD, 2, F0), jnp.float32),
    )(p, dis, W, br)


def _tc_layer256(p, tables, dis, W, b):
    cin = p.shape[0]
    fout = W.shape[1]
    cout2 = fout // 256
    Wr = W.reshape(cin, 2, F0, fout)
    br = b.reshape(1, fout)

    def body(p_ref, t_ref, dis_ref, w_ref, b_ref, out_ref):
        d = dis_ref[...]
        acc = jnp.zeros((RB2, fout), jnp.float32)
        for cc in range(cin):
            for hh in range(2):
                y = (p_ref[cc, 0, :, hh, :] + t_ref[cc, :, hh, :]) * d
                acc = acc + jnp.dot(y, w_ref[cc, hh],
                                    preferred_element_type=jnp.float32)
        h = jnp.maximum(acc + b_ref[...], 0.0) * d[:, 0:1]
        for k in range(cout2):
            out_ref[k, :, 0, :] = h[:, k * 256:k * 256 + F0]
            out_ref[k, :, 1, :] = h[:, k * 256 + F0:(k + 1) * 256]

    return pl.pallas_call(
        body, grid=(HALF // RB2 * 2,),
        in_specs=[pl.BlockSpec((cin, 1, RB2, 2, F0),
                               lambda i: (0, i // 4, i % 4, 0, 0)),
                  pl.BlockSpec((cin, RB2, 2, F0), lambda i: (0, i, 0, 0)),
                  pl.BlockSpec((RB2, F0), lambda i: (i, 0)),
                  pl.BlockSpec((cin, 2, F0, fout), lambda i: (0, 0, 0, 0)),
                  pl.BlockSpec((1, fout), lambda i: (0, 0))],
        out_specs=pl.BlockSpec((cout2, RB2, 2, F0), lambda i: (0, i, 0, 0)),
        out_shape=jax.ShapeDtypeStruct((cout2, NPAD, 2, F0), jnp.float32),
    )(p, tables, dis, Wr, br)


def _tc_final256(p, tables, dis, Wmu, bmu, Wstd, bstd):
    cin = p.shape[0]
    fout = Wmu.shape[1]
    Wmur = Wmu.reshape(cin, 2, F0, fout)
    Wstdr = Wstd.reshape(cin, 2, F0, fout)
    bmur = bmu.reshape(1, fout)
    bstdr = bstd.reshape(1, fout)

    def body(p_ref, t_ref, dis_ref, wmu_ref, bmu_ref, wstd_ref, bstd_ref,
             mu_ref, std_ref):
        d = dis_ref[...]
        accmu = jnp.zeros((RB2, fout), jnp.float32)
        accstd = jnp.zeros((RB2, fout), jnp.float32)
        for cc in range(cin):
            for hh in range(2):
                t = (p_ref[cc, 0, :, hh, :] + t_ref[cc, :, hh, :]) * d
                accmu = accmu + jnp.dot(t, wmu_ref[cc, hh],
                                        preferred_element_type=jnp.float32)
                accstd = accstd + jnp.dot(t, wstd_ref[cc, hh],
                                          preferred_element_type=jnp.float32)
        mu_ref[...] = accmu + bmu_ref[...]
        std_ref[...] = accstd + bstd_ref[...]

    return pl.pallas_call(
        body, grid=(HALF // RB2 * 2,),
        in_specs=[pl.BlockSpec((cin, 1, RB2, 2, F0),
                               lambda i: (0, i // 4, i % 4, 0, 0)),
                  pl.BlockSpec((cin, RB2, 2, F0), lambda i: (0, i, 0, 0)),
                  pl.BlockSpec((RB2, F0), lambda i: (i, 0)),
                  pl.BlockSpec((cin, 2, F0, fout), lambda i: (0, 0, 0, 0)),
                  pl.BlockSpec((1, fout), lambda i: (0, 0)),
                  pl.BlockSpec((cin, 2, F0, fout), lambda i: (0, 0, 0, 0)),
                  pl.BlockSpec((1, fout), lambda i: (0, 0))],
        out_specs=[pl.BlockSpec((RB2, fout), lambda i: (i, 0))] * 2,
        out_shape=[jax.ShapeDtypeStruct((NPAD, fout), jnp.float32)] * 2,
    )(p, tables, dis, Wmur, bmur, Wstdr, bstdr)


def _tc_prep(pdeg, v_pad):
    def body(p_ref, v_ref, dis_ref, xs_ref):
        deg = p_ref[0] + p_ref[1]
        d = lax.rsqrt(deg)
        d = d * (1.5 - 0.5 * deg * d * d)
        dis_ref[...] = d
        xs_ref[...] = d * v_ref[...]

    return pl.pallas_call(
        body, grid=(GRID,),
        in_specs=[pl.BlockSpec((2, RB, F0), lambda i: (0, i, 0)),
                  pl.BlockSpec((RB, F0), lambda i: (i, 0))],
        out_specs=[pl.BlockSpec((RB, F0), lambda i: (i, 0))] * 2,
        out_shape=[jax.ShapeDtypeStruct((NPAD, F0), jnp.float32)] * 2,
    )(pdeg, v_pad)


def _tc_layer(p, dis, W, b):
    cin = p.shape[0]
    fout = W.shape[1]
    cout = fout // F0
    Wr = W.reshape(cin, F0, fout)
    br = b.reshape(1, fout)

    def body(p_ref, dis_ref, w_ref, b_ref, out_ref):
        d = dis_ref[...]
        acc = jnp.zeros((RB, fout), jnp.float32)
        for cc in range(cin):
            y = (p_ref[cc, 0] + p_ref[cc, 1]) * d
            acc = acc + jnp.dot(y, w_ref[cc], preferred_element_type=jnp.float32)
        h = jnp.maximum(acc + b_ref[...], 0.0) * d[:, 0:1]
        for k in range(cout):
            out_ref[k] = h[:, k * F0:(k + 1) * F0]

    return pl.pallas_call(
        body, grid=(GRID,),
        in_specs=[pl.BlockSpec((cin, 2, RB, F0), lambda i: (0, 0, i, 0)),
                  pl.BlockSpec((RB, F0), lambda i: (i, 0)),
                  pl.BlockSpec((cin, F0, fout), lambda i: (0, 0, 0)),
                  pl.BlockSpec((1, fout), lambda i: (0, 0))],
        out_specs=pl.BlockSpec((cout, RB, F0), lambda i: (0, i, 0)),
        out_shape=jax.ShapeDtypeStruct((cout, NPAD, F0), jnp.float32),
    )(p, dis, Wr, br)


def _tc_final(p, dis, Wmu, bmu, Wstd, bstd):
    cin = p.shape[0]
    fout = Wmu.shape[1]
    Wmur = Wmu.reshape(cin, F0, fout)
    Wstdr = Wstd.reshape(cin, F0, fout)
    bmur = bmu.reshape(1, fout)
    bstdr = bstd.reshape(1, fout)

    def body(p_ref, dis_ref, wmu_ref, bmu_ref, wstd_ref, bstd_ref,
             mu_ref, std_ref):
        d = dis_ref[...]
        accmu = jnp.zeros((RB, fout), jnp.float32)
        accstd = jnp.zeros((RB, fout), jnp.float32)
        for cc in range(cin):
            t = (p_ref[cc, 0] + p_ref[cc, 1]) * d
            accmu = accmu + jnp.dot(t, wmu_ref[cc], preferred_element_type=jnp.float32)
            accstd = accstd + jnp.dot(t, wstd_ref[cc], preferred_element_type=jnp.float32)
        mu_ref[...] = accmu + bmu_ref[...]
        std_ref[...] = accstd + bstd_ref[...]

    rblk = lambda i: (i, 0)
    return pl.pallas_call(
        body, grid=(GRID,),
        in_specs=[pl.BlockSpec((cin, 2, RB, F0), lambda i: (0, 0, i, 0)),
                  pl.BlockSpec((RB, F0), rblk),
                  pl.BlockSpec((cin, F0, fout), lambda i: (0, 0, 0)),
                  pl.BlockSpec((1, fout), lambda i: (0, 0)),
                  pl.BlockSpec((cin, F0, fout), lambda i: (0, 0, 0)),
                  pl.BlockSpec((1, fout), lambda i: (0, 0))],
        out_specs=[pl.BlockSpec((RB, fout), rblk)] * 2,
        out_shape=[jax.ShapeDtypeStruct((NPAD, fout), jnp.float32)] * 2,
    )(p, dis, Wmur, bmur, Wstdr, bstdr)


def kernel(v, edge_index, W1, b1, W2, b2, W3, b3, Wmu, bmu, Wstd, bstd):
    epw = E // NW
    src0 = edge_index[0].reshape(NW, epw)
    dst0 = edge_index[1].reshape(NW, epw)
    pad = EPW - epw
    srcp = jnp.pad(src0, ((0, 0), (0, pad)))
    dstp = jnp.pad(dst0, ((0, 0), (0, pad)), constant_values=N)
    src = srcp.reshape(NW, NBLK, BLK)
    dst = dstp.reshape(NW, NBLK, BLK)
    zeros = jnp.zeros((NPAD, F0), jnp.float32)
    ones = jnp.ones((NPAD, F0), jnp.float32)
    v_pad = jnp.pad(v, ((0, NPAD - N), (0, 0)))

    srcF = srcp
    dstF = dstp
    tsrc = jnp.zeros((2 * EPW,), jnp.int32)
    tdst = jnp.full((2 * EPW,), HALF, jnp.int32)
    zeros2 = jnp.zeros((NR, 2, F0), jnp.float32)

    pdeg = _sc_deg(ones, zeros, dst)
    dis, xs0 = _tc_prep(pdeg, v_pad)
    psrc, pdst, pcnt = _sc_partition(srcF, dstF, tsrc, tdst)
    psrc = psrc.reshape(NW, 2, NBLK2, BLK2)
    pdst = pdst.reshape(NW, 2, NBLK2, BLK2)

    p1 = _sc_spmm(xs0[None], zeros, src, dst)
    t = _tc_layer1(p1, dis, W1, b1)
    for W, b in ((W2, b2), (W3, b3)):
        p = _sc_spmm256(t, zeros2, psrc, pdst, pcnt)
        t = _tc_layer256(p, t, dis, W, b)
    p = _sc_spmm256(t, zeros2, psrc, pdst, pcnt)
    mu, std = _tc_final256(p, t, dis, Wmu, bmu, Wstd, bstd)
    return mu[:N], std[:N]
